# baseline (speedup 1.0000x reference)
import math

import jax
import jax.numpy as jnp
from jax.experimental import pallas as pl
from jax.experimental.pallas import tpu as pltpu

LEN_LAB = 50
LEN_INP = 30
LEN_MED = 40
LEN_ALL = LEN_LAB + LEN_INP + LEN_MED
DIM_LAB = 64
DIM_INP = 32
DIM_MED = 32
D = DIM_LAB + DIM_INP + DIM_MED
D1 = 64
NEG = -1e30


def _mm(x, w):
    return jax.lax.dot_general(x, w, (((1,), (0,)), ((), ())),
                               preferred_element_type=jnp.float32)


def _mmT(x, y):
    return jax.lax.dot_general(x, y, (((0,), (0,)), ((), ())),
                               preferred_element_type=jnp.float32)


def _leaky(x):
    return jnp.where(x > 0, x, 0.01 * x)


def _fused_kernel(tstar_ref, data_ref, decay_ref, mask_ref,
                  Wblk_ref, bcat_ref, nodes0_ref, G0_ref,
                  We0_ref, We1_ref, Wl1_ref, Wi1_ref, Wm1_ref,
                  Wq_ref, Wk_ref, Wv_ref, Wo_ref,
                  Wbeta_ref, bbeta_ref, Wout_ref, bout_ref,
                  out_ref):
    b = pl.program_id(0)
    t = tstar_ref[b]
    dat = data_ref[0]
    lab = dat[:, :LEN_LAB]
    inp_obs = (dat[:, LEN_LAB:LEN_LAB + LEN_INP] != 0.0).astype(jnp.float32)
    med = dat[:, LEN_LAB + LEN_INP:]
    a_l = lab * decay_ref[0]
    A = jnp.concatenate([a_l, inp_obs, med], axis=1)
    M = jnp.concatenate([mask_ref[0], inp_obs, med], axis=1)

    h_e = _mm(M, Wblk_ref[...]) + bcat_ref[...]

    he_t = _mm(h_e, We0_ref[...])
    h_e = _leaky(he_t + _mm(A, G0_ref[...]))
    nodes = _leaky(nodes0_ref[...] + _mmT(A, he_t))

    he_t = _mm(h_e, We1_ref[...])
    G1 = jnp.concatenate([
        _mm(nodes[:LEN_LAB], Wl1_ref[...]),
        _mm(nodes[LEN_LAB:LEN_LAB + LEN_INP], Wi1_ref[...]),
        _mm(nodes[LEN_LAB + LEN_INP:], Wm1_ref[...]),
    ], axis=0)
    nodes = nodes + _mmT(A, he_t)

    T = dat.shape[0]
    onehot = (jax.lax.broadcasted_iota(jnp.int32, (T, 1), 0) == t
              ).astype(jnp.float32)
    he_row = _mmT(onehot, he_t)
    a_row = _mmT(onehot, A)
    h_row = he_row + _mm(a_row, G1)

    q = _mm(h_row, Wq_ref[...])
    K = _mm(nodes, Wk_ref[...])
    V = _mm(nodes, Wv_ref[...])
    s = jax.lax.dot_general(q, K, (((1,), (1,)), ((), ())),
                            preferred_element_type=jnp.float32)
    s = s * (1.0 / math.sqrt(float(D)))
    s = s - jnp.max(s, axis=1, keepdims=True)
    e = jnp.exp(s)
    attn = e / jnp.sum(e, axis=1, keepdims=True)
    ctx = _mm(attn, V)

    h_out = _mm(ctx + h_row, Wo_ref[...])
    beta = jnp.tanh(_mm(h_out, Wbeta_ref[...]) + bbeta_ref[...])
    logit = _mm(beta, Wout_ref[...]) + bout_ref[...]
    logit = logit - jnp.max(logit, axis=1, keepdims=True)
    el = jnp.exp(logit)
    out_ref[...] = (el / jnp.sum(el, axis=1, keepdims=True)).reshape(1, 1, D)


def kernel(data, decay, time, label, lab_mask, length, pid,
           W_lab, b_lab, W_inp, b_inp, W_med, b_med,
           We0, Wl0, Wi0, Wm0, We1, Wl1, Wi1, Wm1,
           Wq, Wk, Wv, Wo, W_beta, b_beta, W_out, b_out):
    B, T, _ = data.shape
    f32 = jnp.float32
    z = jnp.zeros

    Wblk = jnp.concatenate([
        jnp.concatenate([W_lab, z((LEN_LAB, DIM_INP + DIM_MED), f32)], 1),
        jnp.concatenate([z((LEN_INP, DIM_LAB), f32), W_inp,
                         z((LEN_INP, DIM_MED), f32)], 1),
        jnp.concatenate([z((LEN_MED, DIM_LAB + DIM_INP), f32), W_med], 1),
    ], 0)
    bcat = jnp.concatenate([b_lab, b_inp, b_med]).reshape(1, D)

    nb = jnp.concatenate([
        jnp.concatenate([jnp.broadcast_to(b_lab, (LEN_LAB, DIM_LAB)),
                         z((LEN_LAB, DIM_INP + DIM_MED), f32)], 1),
        jnp.concatenate([z((LEN_INP, DIM_LAB), f32),
                         jnp.broadcast_to(b_inp, (LEN_INP, DIM_INP)),
                         z((LEN_INP, DIM_MED), f32)], 1),
        jnp.concatenate([z((LEN_MED, DIM_LAB + DIM_INP), f32),
                         jnp.broadcast_to(b_med, (LEN_MED, DIM_MED))], 1),
    ], 0)
    nodes0 = Wblk + nb
    G0 = jnp.concatenate([
        nodes0[:LEN_LAB] @ Wl0,
        nodes0[LEN_LAB:LEN_LAB + LEN_INP] @ Wi0,
        nodes0[LEN_LAB + LEN_INP:] @ Wm0,
    ], 0)

    tstar = (jnp.clip(length, 1, T) - 1).astype(jnp.int32)
    Wout_pad = jnp.concatenate([W_out, z((D1, D - 2), f32)], 1)
    bout_pad = jnp.concatenate([b_out, jnp.full((D - 2,), NEG, f32)]
                               ).reshape(1, D)
    bbeta = b_beta.reshape(1, D1)

    full = lambda shape: pl.BlockSpec(shape, lambda i, s: (0,) * len(shape))
    grid_spec = pltpu.PrefetchScalarGridSpec(
        num_scalar_prefetch=1,
        grid=(B,),
        in_specs=[
            pl.BlockSpec((1, T, LEN_ALL), lambda i, s: (i, 0, 0)),
            pl.BlockSpec((1, T, LEN_LAB), lambda i, s: (i, 0, 0)),
            pl.BlockSpec((1, T, LEN_LAB), lambda i, s: (i, 0, 0)),
            full((LEN_ALL, D)), full((1, D)), full((LEN_ALL, D)),
            full((LEN_ALL, D)),
            full((D, D)), full((D, D)), full((D, D)), full((D, D)),
            full((D, D)),
            full((D, D)), full((D, D)), full((D, D)), full((D, D)),
            full((D, D1)), full((1, D1)), full((D1, D)), full((1, D)),
        ],
        out_specs=pl.BlockSpec((1, 1, D), lambda i, s: (i, 0, 0)),
    )
    out = pl.pallas_call(
        _fused_kernel,
        grid_spec=grid_spec,
        out_shape=jax.ShapeDtypeStruct((B, 1, D), f32),
    )(tstar, data, decay, lab_mask, Wblk, bcat, nodes0, G0,
      We0, We1, Wl1, Wi1, Wm1, Wq, Wk, Wv, Wo,
      W_beta, bbeta, Wout_pad, bout_pad)
    return (out[:, 0, :2], label)


# speedup vs baseline: 2.0275x; 2.0275x over previous
import math

import jax
import jax.numpy as jnp
from jax.experimental import pallas as pl
from jax.experimental.pallas import tpu as pltpu

LEN_LAB = 50
LEN_INP = 30
LEN_MED = 40
LEN_ALL = LEN_LAB + LEN_INP + LEN_MED
DIM_LAB = 64
DIM_INP = 32
DIM_MED = 32
D = DIM_LAB + DIM_INP + DIM_MED
D1 = 64
NEG = -1e30
BB = 8


def _mm(x, w):
    return jax.lax.dot_general(x, w, (((1,), (0,)), ((), ())),
                               preferred_element_type=jnp.float32)


def _mmT(x, y):
    return jax.lax.dot_general(x, y, (((0,), (0,)), ((), ())),
                               preferred_element_type=jnp.float32)


def _mmR(x, y):
    return jax.lax.dot_general(x, y, (((1,), (1,)), ((), ())),
                               preferred_element_type=jnp.float32)


def _leaky(x):
    return jnp.where(x > 0, x, 0.01 * x)


def _fused_kernel(tstar_ref, data_ref, decay_ref, mask_ref,
                  Wblk_ref, bcat_ref, nodes0_ref, G0_ref,
                  We0_ref, We1_ref, Wl1_ref, Wi1_ref, Wm1_ref,
                  Wq_ref, Wk_ref, Wv_ref, Wo_ref,
                  Wbeta_ref, bbeta_ref, Wout_ref, bout_ref,
                  out_ref):
    g = pl.program_id(0)
    f32 = jnp.float32
    dat = data_ref[...]
    T = dat.shape[1]
    flat = dat.reshape(BB * T, LEN_ALL)
    lab = flat[:, :LEN_LAB]
    inp_obs = (flat[:, LEN_LAB:LEN_LAB + LEN_INP] != 0.0).astype(f32)
    med = flat[:, LEN_LAB + LEN_INP:]
    a_l = lab * decay_ref[...].reshape(BB * T, LEN_LAB)
    A = jnp.concatenate([a_l, inp_obs, med], axis=1)
    M = jnp.concatenate([mask_ref[...].reshape(BB * T, LEN_LAB),
                         inp_obs, med], axis=1)

    h_e = _mm(M, Wblk_ref[...]) + bcat_ref[...]

    he_t0 = _mm(h_e, We0_ref[...])
    h_e = _leaky(he_t0 + _mm(A, G0_ref[...]))
    nodes0 = nodes0_ref[...]
    nodes = [
        _leaky(nodes0 + _mmT(A[b * T:(b + 1) * T], he_t0[b * T:(b + 1) * T]))
        for b in range(BB)
    ]

    he_t1 = _mm(h_e, We1_ref[...])

    lane = jax.lax.broadcasted_iota(jnp.int32, (1, LEN_ALL), 1)
    msk_l = (lane < LEN_LAB).astype(f32)
    msk_i = ((lane >= LEN_LAB) & (lane < LEN_LAB + LEN_INP)).astype(f32)
    msk_m = (lane >= LEN_LAB + LEN_INP).astype(f32)

    rows_h = []
    nodes1 = []
    for b in range(BB):
        t = tstar_ref[g * BB + b]
        sl = slice(b * T, (b + 1) * T)
        nodes1.append(nodes[b] + _mmT(A[sl], he_t1[sl]))
        onehot = (jax.lax.broadcasted_iota(jnp.int32, (T, 1), 0) == t
                  ).astype(f32)
        he_row = _mmT(onehot, he_t1[sl])
        a_row = _mmT(onehot, A[sl])
        msg = (_mm(_mm(a_row * msk_l, nodes[b]), Wl1_ref[...])
               + _mm(_mm(a_row * msk_i, nodes[b]), Wi1_ref[...])
               + _mm(_mm(a_row * msk_m, nodes[b]), Wm1_ref[...]))
        rows_h.append(he_row + msg)

    Nall = jnp.concatenate(nodes1, axis=0)
    K = _mm(Nall, Wk_ref[...])
    V = _mm(Nall, Wv_ref[...])
    inv_sqrt_d = 1.0 / math.sqrt(float(D))
    ctxs = []
    for b in range(BB):
        q = _mm(rows_h[b], Wq_ref[...])
        Kb = K[b * LEN_ALL:(b + 1) * LEN_ALL]
        s = _mmR(q, Kb) * inv_sqrt_d
        s = s - jnp.max(s, axis=1, keepdims=True)
        e = jnp.exp(s)
        attn = e / jnp.sum(e, axis=1, keepdims=True)
        ctxs.append(_mm(attn, V[b * LEN_ALL:(b + 1) * LEN_ALL]))

    H = jnp.concatenate(rows_h, axis=0)
    C = jnp.concatenate(ctxs, axis=0)
    h_out = _mm(C + H, Wo_ref[...])
    beta = jnp.tanh(_mm(h_out, Wbeta_ref[...]) + bbeta_ref[...])
    logit = _mm(beta, Wout_ref[...]) + bout_ref[...]
    logit = logit - jnp.max(logit, axis=1, keepdims=True)
    el = jnp.exp(logit)
    p = el / jnp.sum(el, axis=1, keepdims=True)
    out_ref[...] = p.reshape(BB, 1, D)


def kernel(data, decay, time, label, lab_mask, length, pid,
           W_lab, b_lab, W_inp, b_inp, W_med, b_med,
           We0, Wl0, Wi0, Wm0, We1, Wl1, Wi1, Wm1,
           Wq, Wk, Wv, Wo, W_beta, b_beta, W_out, b_out):
    B, T, _ = data.shape
    f32 = jnp.float32
    z = jnp.zeros

    Wblk = jnp.concatenate([
        jnp.concatenate([W_lab, z((LEN_LAB, DIM_INP + DIM_MED), f32)], 1),
        jnp.concatenate([z((LEN_INP, DIM_LAB), f32), W_inp,
                         z((LEN_INP, DIM_MED), f32)], 1),
        jnp.concatenate([z((LEN_MED, DIM_LAB + DIM_INP), f32), W_med], 1),
    ], 0)
    bcat = jnp.concatenate([b_lab, b_inp, b_med]).reshape(1, D)

    nb = jnp.concatenate([
        jnp.concatenate([jnp.broadcast_to(b_lab, (LEN_LAB, DIM_LAB)),
                         z((LEN_LAB, DIM_INP + DIM_MED), f32)], 1),
        jnp.concatenate([z((LEN_INP, DIM_LAB), f32),
                         jnp.broadcast_to(b_inp, (LEN_INP, DIM_INP)),
                         z((LEN_INP, DIM_MED), f32)], 1),
        jnp.concatenate([z((LEN_MED, DIM_LAB + DIM_INP), f32),
                         jnp.broadcast_to(b_med, (LEN_MED, DIM_MED))], 1),
    ], 0)
    nodes0 = Wblk + nb
    G0 = jnp.concatenate([
        nodes0[:LEN_LAB] @ Wl0,
        nodes0[LEN_LAB:LEN_LAB + LEN_INP] @ Wi0,
        nodes0[LEN_LAB + LEN_INP:] @ Wm0,
    ], 0)

    tstar = (jnp.clip(length, 1, T) - 1).astype(jnp.int32)
    Wout_pad = jnp.concatenate([W_out, z((D1, D - 2), f32)], 1)
    bout_pad = jnp.concatenate([b_out, jnp.full((D - 2,), NEG, f32)]
                               ).reshape(1, D)
    bbeta = b_beta.reshape(1, D1)

    full = lambda shape: pl.BlockSpec(shape, lambda i, s: (0,) * len(shape))
    grid_spec = pltpu.PrefetchScalarGridSpec(
        num_scalar_prefetch=1,
        grid=(B // BB,),
        in_specs=[
            pl.BlockSpec((BB, T, LEN_ALL), lambda i, s: (i, 0, 0)),
            pl.BlockSpec((BB, T, LEN_LAB), lambda i, s: (i, 0, 0)),
            pl.BlockSpec((BB, T, LEN_LAB), lambda i, s: (i, 0, 0)),
            full((LEN_ALL, D)), full((1, D)), full((LEN_ALL, D)),
            full((LEN_ALL, D)),
            full((D, D)), full((D, D)), full((D, D)), full((D, D)),
            full((D, D)),
            full((D, D)), full((D, D)), full((D, D)), full((D, D)),
            full((D, D1)), full((1, D1)), full((D1, D)), full((1, D)),
        ],
        out_specs=pl.BlockSpec((BB, 1, D), lambda i, s: (i, 0, 0)),
    )
    out = pl.pallas_call(
        _fused_kernel,
        grid_spec=grid_spec,
        out_shape=jax.ShapeDtypeStruct((B, 1, D), f32),
    )(tstar, data, decay, lab_mask, Wblk, bcat, nodes0, G0,
      We0, We1, Wl1, Wi1, Wm1, Wq, Wk, Wv, Wo,
      W_beta, bbeta, Wout_pad, bout_pad)
    return (out[:, 0, :2], label)


# speedup vs baseline: 3.3340x; 1.6444x over previous
import math

import jax
import jax.numpy as jnp
from jax.experimental import pallas as pl
from jax.experimental.pallas import tpu as pltpu

LEN_LAB = 50
LEN_INP = 30
LEN_MED = 40
LEN_ALL = LEN_LAB + LEN_INP + LEN_MED
DIM_LAB = 64
DIM_INP = 32
DIM_MED = 32
D = DIM_LAB + DIM_INP + DIM_MED
D1 = 64
NEG = -1e30
BB = 8


def _mm(x, w):
    return jax.lax.dot_general(x, w, (((1,), (0,)), ((), ())),
                               preferred_element_type=jnp.float32)


def _mmT(x, y):
    return jax.lax.dot_general(x, y, (((0,), (0,)), ((), ())),
                               preferred_element_type=jnp.float32)


def _mmR(x, y):
    return jax.lax.dot_general(x, y, (((1,), (1,)), ((), ())),
                               preferred_element_type=jnp.float32)


def _leaky(x):
    return jnp.where(x > 0, x, 0.01 * x)


def _fused_kernel(tstar_ref, data_ref, decay_ref, mask_ref,
                  Wblk_ref, bcat_ref, nodes0_ref, G0_ref,
                  We0_ref, We1_ref, Wl1_ref, Wi1_ref, Wm1_ref,
                  Wq_ref, Wk_ref, Wv_ref, Wo_ref,
                  Wbeta_ref, bbeta_ref, Wout_ref, bout_ref,
                  out_ref):
    g = pl.program_id(0)
    f32 = jnp.float32
    dat = data_ref[...]
    T = dat.shape[1]
    flat = dat.reshape(BB * T, LEN_ALL)
    lab = flat[:, :LEN_LAB]
    inp_obs = (flat[:, LEN_LAB:LEN_LAB + LEN_INP] != 0.0).astype(f32)
    med = flat[:, LEN_LAB + LEN_INP:]
    a_l = lab * decay_ref[...].reshape(BB * T, LEN_LAB)
    A = jnp.concatenate([a_l, inp_obs, med], axis=1)
    M = jnp.concatenate([mask_ref[...].reshape(BB * T, LEN_LAB),
                         inp_obs, med], axis=1)

    h_e = _mm(M, Wblk_ref[...]) + bcat_ref[...]

    he_t0 = _mm(h_e, We0_ref[...])
    h_e = _leaky(he_t0 + _mm(A, G0_ref[...]))
    nodes0 = nodes0_ref[...]
    nodes = [
        _leaky(nodes0 + _mmT(A[b * T:(b + 1) * T], he_t0[b * T:(b + 1) * T]))
        for b in range(BB)
    ]

    he_t1 = _mm(h_e, We1_ref[...])
    nodes1 = [nodes[b] + _mmT(A[b * T:(b + 1) * T], he_t1[b * T:(b + 1) * T])
              for b in range(BB)]

    tcol = jnp.concatenate(
        [jnp.full((1, 1), tstar_ref[g * BB + b], jnp.int32) for b in range(BB)],
        axis=0)
    ri = jax.lax.broadcasted_iota(jnp.int32, (BB, BB * T), 0)
    li = jax.lax.broadcasted_iota(jnp.int32, (BB, BB * T), 1)
    OH = (li == ri * T + tcol).astype(f32)
    he_rows = _mm(OH, he_t1)
    a_rows = _mm(OH, A)

    N0 = jnp.concatenate(nodes, axis=0)
    rs = jax.lax.broadcasted_iota(jnp.int32, (BB, BB * LEN_ALL), 0)
    ls = jax.lax.broadcasted_iota(jnp.int32, (BB, BB * LEN_ALL), 1)
    off = ls - rs * LEN_ALL
    strip = (off >= 0) & (off < LEN_ALL)
    strip_l = (strip & (off < LEN_LAB)).astype(f32)
    strip_i = (strip & (off >= LEN_LAB)
               & (off < LEN_LAB + LEN_INP)).astype(f32)
    strip_m = (strip & (off >= LEN_LAB + LEN_INP)).astype(f32)
    a_tiled = jnp.concatenate([a_rows] * BB, axis=1)
    msg = (_mm(_mm(a_tiled * strip_l, N0), Wl1_ref[...])
           + _mm(_mm(a_tiled * strip_i, N0), Wi1_ref[...])
           + _mm(_mm(a_tiled * strip_m, N0), Wm1_ref[...]))
    H = he_rows + msg

    Nall = jnp.concatenate(nodes1, axis=0)
    K = _mm(Nall, Wk_ref[...])
    V = _mm(Nall, Wv_ref[...])
    Q = _mm(H, Wq_ref[...])
    inv_sqrt_d = 1.0 / math.sqrt(float(D))
    S = _mmR(Q, K) * inv_sqrt_d
    S = jnp.where(strip, S, NEG)
    S = S - jnp.max(S, axis=1, keepdims=True)
    E = jnp.where(strip, jnp.exp(S), 0.0)
    attn = E / jnp.sum(E, axis=1, keepdims=True)
    C = _mm(attn, V)

    h_out = _mm(C + H, Wo_ref[...])
    beta = jnp.tanh(_mm(h_out, Wbeta_ref[...]) + bbeta_ref[...])
    logit = _mm(beta, Wout_ref[...]) + bout_ref[...]
    logit = logit - jnp.max(logit, axis=1, keepdims=True)
    el = jnp.exp(logit)
    p = el / jnp.sum(el, axis=1, keepdims=True)
    out_ref[...] = p.reshape(BB, 1, D)


def kernel(data, decay, time, label, lab_mask, length, pid,
           W_lab, b_lab, W_inp, b_inp, W_med, b_med,
           We0, Wl0, Wi0, Wm0, We1, Wl1, Wi1, Wm1,
           Wq, Wk, Wv, Wo, W_beta, b_beta, W_out, b_out):
    B, T, _ = data.shape
    f32 = jnp.float32
    z = jnp.zeros

    Wblk = jnp.concatenate([
        jnp.concatenate([W_lab, z((LEN_LAB, DIM_INP + DIM_MED), f32)], 1),
        jnp.concatenate([z((LEN_INP, DIM_LAB), f32), W_inp,
                         z((LEN_INP, DIM_MED), f32)], 1),
        jnp.concatenate([z((LEN_MED, DIM_LAB + DIM_INP), f32), W_med], 1),
    ], 0)
    bcat = jnp.concatenate([b_lab, b_inp, b_med]).reshape(1, D)

    nb = jnp.concatenate([
        jnp.concatenate([jnp.broadcast_to(b_lab, (LEN_LAB, DIM_LAB)),
                         z((LEN_LAB, DIM_INP + DIM_MED), f32)], 1),
        jnp.concatenate([z((LEN_INP, DIM_LAB), f32),
                         jnp.broadcast_to(b_inp, (LEN_INP, DIM_INP)),
                         z((LEN_INP, DIM_MED), f32)], 1),
        jnp.concatenate([z((LEN_MED, DIM_LAB + DIM_INP), f32),
                         jnp.broadcast_to(b_med, (LEN_MED, DIM_MED))], 1),
    ], 0)
    nodes0 = Wblk + nb
    G0 = jnp.concatenate([
        nodes0[:LEN_LAB] @ Wl0,
        nodes0[LEN_LAB:LEN_LAB + LEN_INP] @ Wi0,
        nodes0[LEN_LAB + LEN_INP:] @ Wm0,
    ], 0)

    tstar = (jnp.clip(length, 1, T) - 1).astype(jnp.int32)
    Wout_pad = jnp.concatenate([W_out, z((D1, D - 2), f32)], 1)
    bout_pad = jnp.concatenate([b_out, jnp.full((D - 2,), NEG, f32)]
                               ).reshape(1, D)
    bbeta = b_beta.reshape(1, D1)

    full = lambda shape: pl.BlockSpec(shape, lambda i, s: (0,) * len(shape))
    grid_spec = pltpu.PrefetchScalarGridSpec(
        num_scalar_prefetch=1,
        grid=(B // BB,),
        in_specs=[
            pl.BlockSpec((BB, T, LEN_ALL), lambda i, s: (i, 0, 0)),
            pl.BlockSpec((BB, T, LEN_LAB), lambda i, s: (i, 0, 0)),
            pl.BlockSpec((BB, T, LEN_LAB), lambda i, s: (i, 0, 0)),
            full((LEN_ALL, D)), full((1, D)), full((LEN_ALL, D)),
            full((LEN_ALL, D)),
            full((D, D)), full((D, D)), full((D, D)), full((D, D)),
            full((D, D)),
            full((D, D)), full((D, D)), full((D, D)), full((D, D)),
            full((D, D1)), full((1, D1)), full((D1, D)), full((1, D)),
        ],
        out_specs=pl.BlockSpec((BB, 1, D), lambda i, s: (i, 0, 0)),
    )
    out = pl.pallas_call(
        _fused_kernel,
        grid_spec=grid_spec,
        out_shape=jax.ShapeDtypeStruct((B, 1, D), f32),
    )(tstar, data, decay, lab_mask, Wblk, bcat, nodes0, G0,
      We0, We1, Wl1, Wi1, Wm1, Wq, Wk, Wv, Wo,
      W_beta, bbeta, Wout_pad, bout_pad)
    return (out[:, 0, :2], label)


# speedup vs baseline: 4.0846x; 1.2251x over previous
import math

import jax
import jax.numpy as jnp
from jax.experimental import pallas as pl
from jax.experimental.pallas import tpu as pltpu

LEN_LAB = 50
LEN_INP = 30
LEN_MED = 40
LEN_ALL = LEN_LAB + LEN_INP + LEN_MED
DIM_LAB = 64
DIM_INP = 32
DIM_MED = 32
D = DIM_LAB + DIM_INP + DIM_MED
D1 = 64
NEG = -1e30
BB = 16


def _mm(x, w):
    return jax.lax.dot_general(x, w, (((1,), (0,)), ((), ())),
                               preferred_element_type=jnp.float32)


def _mmT(x, y):
    return jax.lax.dot_general(x, y, (((0,), (0,)), ((), ())),
                               preferred_element_type=jnp.float32)


def _mmR(x, y):
    return jax.lax.dot_general(x, y, (((1,), (1,)), ((), ())),
                               preferred_element_type=jnp.float32)


def _leaky(x):
    return jnp.where(x > 0, x, 0.01 * x)


def _fused_kernel(tstar_ref, data_ref, decay_ref, mask_ref,
                  Wblk_ref, bcat_ref, nodes0_ref, G0_ref,
                  We0_ref, We1_ref, Wl1_ref, Wi1_ref, Wm1_ref,
                  Wq_ref, Wk_ref, Wv_ref, Wo_ref,
                  Wbeta_ref, bbeta_ref, Wout_ref, bout_ref,
                  out_ref):
    g = pl.program_id(0)
    f32 = jnp.float32
    dat = data_ref[...]
    T = dat.shape[1]
    flat = dat.reshape(BB * T, LEN_ALL)
    lab = flat[:, :LEN_LAB]
    inp_obs = (flat[:, LEN_LAB:LEN_LAB + LEN_INP] != 0.0).astype(f32)
    med = flat[:, LEN_LAB + LEN_INP:]
    a_l = lab * decay_ref[...].reshape(BB * T, LEN_LAB)
    A = jnp.concatenate([a_l, inp_obs, med], axis=1)
    M = jnp.concatenate([mask_ref[...].reshape(BB * T, LEN_LAB),
                         inp_obs, med], axis=1)

    h_e = _mm(M, Wblk_ref[...]) + bcat_ref[...]

    he_t0 = _mm(h_e, We0_ref[...])
    h_e = _leaky(he_t0 + _mm(A, G0_ref[...]))
    nodes0 = nodes0_ref[...]
    nodes = [
        _leaky(nodes0 + _mmT(A[b * T:(b + 1) * T], he_t0[b * T:(b + 1) * T]))
        for b in range(BB)
    ]

    he_t1 = _mm(h_e, We1_ref[...])
    nodes1 = [nodes[b] + _mmT(A[b * T:(b + 1) * T], he_t1[b * T:(b + 1) * T])
              for b in range(BB)]

    tcol = jnp.concatenate(
        [jnp.full((1, 1), tstar_ref[g * BB + b], jnp.int32) for b in range(BB)],
        axis=0)
    ri = jax.lax.broadcasted_iota(jnp.int32, (BB, BB * T), 0)
    li = jax.lax.broadcasted_iota(jnp.int32, (BB, BB * T), 1)
    OH = (li == ri * T + tcol).astype(f32)
    he_rows = _mm(OH, he_t1)
    a_rows = _mm(OH, A)

    N0 = jnp.concatenate(nodes, axis=0)
    rs = jax.lax.broadcasted_iota(jnp.int32, (BB, BB * LEN_ALL), 0)
    ls = jax.lax.broadcasted_iota(jnp.int32, (BB, BB * LEN_ALL), 1)
    off = ls - rs * LEN_ALL
    strip = (off >= 0) & (off < LEN_ALL)
    strip_l = (strip & (off < LEN_LAB)).astype(f32)
    strip_i = (strip & (off >= LEN_LAB)
               & (off < LEN_LAB + LEN_INP)).astype(f32)
    strip_m = (strip & (off >= LEN_LAB + LEN_INP)).astype(f32)
    a_tiled = jnp.concatenate([a_rows] * BB, axis=1)
    msg = (_mm(_mm(a_tiled * strip_l, N0), Wl1_ref[...])
           + _mm(_mm(a_tiled * strip_i, N0), Wi1_ref[...])
           + _mm(_mm(a_tiled * strip_m, N0), Wm1_ref[...]))
    H = he_rows + msg

    Nall = jnp.concatenate(nodes1, axis=0)
    K = _mm(Nall, Wk_ref[...])
    V = _mm(Nall, Wv_ref[...])
    Q = _mm(H, Wq_ref[...])
    inv_sqrt_d = 1.0 / math.sqrt(float(D))
    S = _mmR(Q, K) * inv_sqrt_d
    S = jnp.where(strip, S, NEG)
    S = S - jnp.max(S, axis=1, keepdims=True)
    E = jnp.where(strip, jnp.exp(S), 0.0)
    attn = E / jnp.sum(E, axis=1, keepdims=True)
    C = _mm(attn, V)

    h_out = _mm(C + H, Wo_ref[...])
    beta = jnp.tanh(_mm(h_out, Wbeta_ref[...]) + bbeta_ref[...])
    logit = _mm(beta, Wout_ref[...]) + bout_ref[...]
    logit = logit - jnp.max(logit, axis=1, keepdims=True)
    el = jnp.exp(logit)
    p = el / jnp.sum(el, axis=1, keepdims=True)
    out_ref[...] = p.reshape(BB, 1, D)


def kernel(data, decay, time, label, lab_mask, length, pid,
           W_lab, b_lab, W_inp, b_inp, W_med, b_med,
           We0, Wl0, Wi0, Wm0, We1, Wl1, Wi1, Wm1,
           Wq, Wk, Wv, Wo, W_beta, b_beta, W_out, b_out):
    B, T, _ = data.shape
    f32 = jnp.float32
    z = jnp.zeros

    Wblk = jnp.concatenate([
        jnp.concatenate([W_lab, z((LEN_LAB, DIM_INP + DIM_MED), f32)], 1),
        jnp.concatenate([z((LEN_INP, DIM_LAB), f32), W_inp,
                         z((LEN_INP, DIM_MED), f32)], 1),
        jnp.concatenate([z((LEN_MED, DIM_LAB + DIM_INP), f32), W_med], 1),
    ], 0)
    bcat = jnp.concatenate([b_lab, b_inp, b_med]).reshape(1, D)

    nb = jnp.concatenate([
        jnp.concatenate([jnp.broadcast_to(b_lab, (LEN_LAB, DIM_LAB)),
                         z((LEN_LAB, DIM_INP + DIM_MED), f32)], 1),
        jnp.concatenate([z((LEN_INP, DIM_LAB), f32),
                         jnp.broadcast_to(b_inp, (LEN_INP, DIM_INP)),
                         z((LEN_INP, DIM_MED), f32)], 1),
        jnp.concatenate([z((LEN_MED, DIM_LAB + DIM_INP), f32),
                         jnp.broadcast_to(b_med, (LEN_MED, DIM_MED))], 1),
    ], 0)
    nodes0 = Wblk + nb
    G0 = jnp.concatenate([
        nodes0[:LEN_LAB] @ Wl0,
        nodes0[LEN_LAB:LEN_LAB + LEN_INP] @ Wi0,
        nodes0[LEN_LAB + LEN_INP:] @ Wm0,
    ], 0)

    tstar = (jnp.clip(length, 1, T) - 1).astype(jnp.int32)
    Wout_pad = jnp.concatenate([W_out, z((D1, D - 2), f32)], 1)
    bout_pad = jnp.concatenate([b_out, jnp.full((D - 2,), NEG, f32)]
                               ).reshape(1, D)
    bbeta = b_beta.reshape(1, D1)

    full = lambda shape: pl.BlockSpec(shape, lambda i, s: (0,) * len(shape))
    grid_spec = pltpu.PrefetchScalarGridSpec(
        num_scalar_prefetch=1,
        grid=(B // BB,),
        in_specs=[
            pl.BlockSpec((BB, T, LEN_ALL), lambda i, s: (i, 0, 0)),
            pl.BlockSpec((BB, T, LEN_LAB), lambda i, s: (i, 0, 0)),
            pl.BlockSpec((BB, T, LEN_LAB), lambda i, s: (i, 0, 0)),
            full((LEN_ALL, D)), full((1, D)), full((LEN_ALL, D)),
            full((LEN_ALL, D)),
            full((D, D)), full((D, D)), full((D, D)), full((D, D)),
            full((D, D)),
            full((D, D)), full((D, D)), full((D, D)), full((D, D)),
            full((D, D1)), full((1, D1)), full((D1, D)), full((1, D)),
        ],
        out_specs=pl.BlockSpec((BB, 1, D), lambda i, s: (i, 0, 0)),
    )
    out = pl.pallas_call(
        _fused_kernel,
        grid_spec=grid_spec,
        out_shape=jax.ShapeDtypeStruct((B, 1, D), f32),
    )(tstar, data, decay, lab_mask, Wblk, bcat, nodes0, G0,
      We0, We1, Wl1, Wi1, Wm1, Wq, Wk, Wv, Wo,
      W_beta, bbeta, Wout_pad, bout_pad)
    return (out[:, 0, :2], label)
